# baseline (speedup 1.0000x reference)
import functools

import jax
import jax.numpy as jnp
from jax import lax
from jax.experimental import pallas as pl
from jax.experimental.pallas import tpu as pltpu
from jax.experimental.pallas import tpu_sc as plsc

D_MODEL = 64
SCALE = 8.0

NUM_CORES = 2
NUM_SUBCORES = 16
NUM_WORKERS = NUM_CORES * NUM_SUBCORES

CHUNK = 256
LANES = 16


def _sc_gather_scale(idx_half, idx_off, table_pairs):
    batch = idx_half.shape[0]
    d2 = table_pairs.shape[1]
    b_per_w = batch // NUM_WORKERS
    n_chunks = b_per_w // CHUNK

    mesh = plsc.VectorSubcoreMesh(core_axis_name="c", subcore_axis_name="s")

    @functools.partial(
        pl.kernel,
        out_type=jax.ShapeDtypeStruct((batch, D_MODEL), jnp.float32),
        mesh=mesh,
        scratch_types=[
            pltpu.VMEM((CHUNK,), jnp.int32),
            pltpu.VMEM((CHUNK,), jnp.int32),
            pltpu.VMEM((CHUNK, 128), jnp.float32),
            pltpu.VMEM((CHUNK, D_MODEL), jnp.float32),
            pltpu.SemaphoreType.DMA,
        ],
    )
    def k(half_hbm, off_hbm, table_hbm, out_hbm, half_v, off_v, pairs_v,
          out_v, sem):
        wid = lax.axis_index("s") * NUM_CORES + lax.axis_index("c")
        base = wid * b_per_w

        @pl.loop(0, n_chunks)
        def _chunk_loop(g):
            off = base + g * CHUNK
            pltpu.sync_copy(half_hbm.at[pl.ds(off, CHUNK)], half_v)
            pltpu.sync_copy(off_hbm.at[pl.ds(off, CHUNK)], off_v)
            pltpu.async_copy(table_hbm.at[half_v], pairs_v, sem).wait()

            @pl.loop(0, CHUNK // LANES)
            def _row_loop(h):
                i16 = h * LANES
                pv = off_v[pl.ds(i16, LANES)]
                for l in range(LANES):
                    p = pv[l]
                    for j in range(D_MODEL // LANES):
                        v = pairs_v.at[i16 + l, pl.ds(p + j * LANES, LANES)][...]
                        out_v.at[i16 + l, pl.ds(j * LANES, LANES)][...] = v * SCALE

            pltpu.sync_copy(out_v, out_hbm.at[pl.ds(off, CHUNK)])

    return k(idx_half, idx_off, table_pairs)


def kernel(x, table):
    vocab = table.shape[0]
    idx = x.reshape(-1)
    table_pairs = table.reshape(vocab // 2, 2 * D_MODEL)
    out = _sc_gather_scale(idx >> 1, (idx & 1) * D_MODEL, table_pairs)
    return out.reshape(x.shape[0], x.shape[1], D_MODEL)


# speedup vs baseline: 1.1844x; 1.1844x over previous
import functools

import jax
import jax.numpy as jnp
from jax import lax
from jax.experimental import pallas as pl
from jax.experimental.pallas import tpu as pltpu
from jax.experimental.pallas import tpu_sc as plsc

D_MODEL = 64
SCALE = 8.0

NUM_CORES = 2
NUM_SUBCORES = 16
NUM_WORKERS = NUM_CORES * NUM_SUBCORES

CHUNK = 160
NBUF = 2
LANES = 16


def _sc_gather_scale(idx_half, idx_off, table_pairs):
    batch = idx_half.shape[0]
    b_per_w = batch // NUM_WORKERS
    n_chunks = b_per_w // CHUNK
    assert n_chunks % NBUF == 0 and n_chunks >= 2 * NBUF

    mesh = plsc.VectorSubcoreMesh(core_axis_name="c", subcore_axis_name="s")

    @functools.partial(
        pl.kernel,
        out_type=jax.ShapeDtypeStruct((batch, D_MODEL), jnp.float32),
        mesh=mesh,
        scratch_types=[
            [pltpu.VMEM((CHUNK,), jnp.int32) for _ in range(NBUF)],
            [pltpu.VMEM((CHUNK,), jnp.int32) for _ in range(NBUF)],
            [pltpu.VMEM((CHUNK, 128), jnp.float32) for _ in range(NBUF)],
            [pltpu.VMEM((CHUNK, D_MODEL), jnp.float32) for _ in range(NBUF)],
            [pltpu.SemaphoreType.DMA for _ in range(NBUF)],
            [pltpu.SemaphoreType.DMA for _ in range(NBUF)],
        ],
    )
    def k(half_hbm, off_hbm, table_hbm, out_hbm, half_v, off_v, pairs_v,
          out_v, gsem, osem):
        wid = lax.axis_index("s") * NUM_CORES + lax.axis_index("c")
        base = wid * b_per_w

        def load_and_gather(g, b):
            off = base + g * CHUNK
            pltpu.sync_copy(half_hbm.at[pl.ds(off, CHUNK)], half_v[b])
            pltpu.sync_copy(off_hbm.at[pl.ds(off, CHUNK)], off_v[b])
            pltpu.async_copy(table_hbm.at[half_v[b]], pairs_v[b], gsem[b])

        for b in range(NBUF):
            load_and_gather(b, b)

        @pl.loop(0, n_chunks // NBUF)
        def _outer(gg):
            for b in range(NBUF):
                g = gg * NBUF + b
                off = base + g * CHUNK
                pltpu.make_async_copy(
                    table_hbm.at[half_v[b]], pairs_v[b], gsem[b]).wait()
                @pl.when(gg > 0)
                def _():
                    pltpu.make_async_copy(
                        out_v[b], out_hbm.at[pl.ds(off, CHUNK)],
                        osem[b]).wait()

                @pl.loop(0, CHUNK // LANES)
                def _rows(h):
                    i16 = h * LANES
                    pv = off_v[b][pl.ds(i16, LANES)]
                    for l in range(LANES):
                        p = pv[l]
                        for j in range(D_MODEL // LANES):
                            v = pairs_v[b].at[
                                i16 + l, pl.ds(p + j * LANES, LANES)][...]
                            out_v[b].at[
                                i16 + l, pl.ds(j * LANES, LANES)][...] = (
                                    v * SCALE)

                pltpu.async_copy(
                    out_v[b], out_hbm.at[pl.ds(off, CHUNK)], osem[b])

                @pl.when(g + NBUF < n_chunks)
                def _():
                    load_and_gather(g + NBUF, b)

        for b in range(NBUF):
            g = n_chunks - NBUF + b
            off = base + g * CHUNK
            pltpu.make_async_copy(
                out_v[b], out_hbm.at[pl.ds(off, CHUNK)], osem[b]).wait()

    return k(idx_half, idx_off, table_pairs)


def kernel(x, table):
    vocab = table.shape[0]
    idx = x.reshape(-1)
    table_pairs = table.reshape(vocab // 2, 2 * D_MODEL)
    out = _sc_gather_scale(idx >> 1, (idx & 1) * D_MODEL, table_pairs)
    return out.reshape(x.shape[0], x.shape[1], D_MODEL)


# speedup vs baseline: 1.7022x; 1.4372x over previous
import functools

import jax
import jax.numpy as jnp
from jax import lax
from jax.experimental import pallas as pl
from jax.experimental.pallas import tpu as pltpu
from jax.experimental.pallas import tpu_sc as plsc

D_MODEL = 64
SCALE = 8.0

NUM_CORES = 2
NUM_SUBCORES = 16
NUM_WORKERS = NUM_CORES * NUM_SUBCORES

CHUNK = 160
NBUF = 2
LANES = 16


def _sc_gather_scale(idx, table_pad):
    batch = idx.shape[0]
    b_per_w = batch // NUM_WORKERS
    n_chunks = b_per_w // CHUNK
    assert n_chunks % NBUF == 0 and n_chunks >= 2 * NBUF

    mesh = plsc.VectorSubcoreMesh(core_axis_name="c", subcore_axis_name="s")

    @functools.partial(
        pl.kernel,
        out_type=jax.ShapeDtypeStruct((batch, D_MODEL), jnp.float32),
        mesh=mesh,
        scratch_types=[
            [pltpu.VMEM((CHUNK,), jnp.int32) for _ in range(NBUF)],
            [pltpu.VMEM((CHUNK, 128), jnp.float32) for _ in range(NBUF)],
            [pltpu.VMEM((CHUNK, D_MODEL), jnp.float32) for _ in range(NBUF)],
            [pltpu.SemaphoreType.DMA for _ in range(NBUF)],
            [pltpu.SemaphoreType.DMA for _ in range(NBUF)],
        ],
    )
    def k(idx_hbm, table_hbm, out_hbm, idx_v, rows_v, out_v, gsem, osem):
        wid = lax.axis_index("s") * NUM_CORES + lax.axis_index("c")
        base = wid * b_per_w

        def load_and_gather(g, b):
            off = base + g * CHUNK
            pltpu.sync_copy(idx_hbm.at[pl.ds(off, CHUNK)], idx_v[b])
            pltpu.async_copy(table_hbm.at[idx_v[b]], rows_v[b], gsem[b])

        for b in range(NBUF):
            load_and_gather(b, b)

        @pl.loop(0, n_chunks // NBUF)
        def _outer(gg):
            for b in range(NBUF):
                g = gg * NBUF + b
                off = base + g * CHUNK
                pltpu.make_async_copy(
                    table_hbm.at[idx_v[b]], rows_v[b], gsem[b]).wait()
                @pl.when(gg > 0)
                def _():
                    pltpu.make_async_copy(
                        out_v[b], out_hbm.at[pl.ds(off, CHUNK)],
                        osem[b]).wait()

                @pl.loop(0, CHUNK // LANES)
                def _rows(h):
                    i16 = h * LANES
                    for l in range(LANES):
                        for j in range(D_MODEL // LANES):
                            v = rows_v[b].at[
                                i16 + l, pl.ds(j * LANES, LANES)][...]
                            out_v[b].at[
                                i16 + l, pl.ds(j * LANES, LANES)][...] = (
                                    v * SCALE)

                pltpu.async_copy(
                    out_v[b], out_hbm.at[pl.ds(off, CHUNK)], osem[b])

                @pl.when(g + NBUF < n_chunks)
                def _():
                    load_and_gather(g + NBUF, b)

        for b in range(NBUF):
            g = n_chunks - NBUF + b
            off = base + g * CHUNK
            pltpu.make_async_copy(
                out_v[b], out_hbm.at[pl.ds(off, CHUNK)], osem[b]).wait()

    return k(idx, table_pad)


def kernel(x, table):
    idx = x.reshape(-1)
    table_pad = jnp.pad(table, ((0, 0), (0, 128 - D_MODEL)))
    out = _sc_gather_scale(idx, table_pad)
    return out.reshape(x.shape[0], x.shape[1], D_MODEL)
